# baseline (speedup 1.0000x reference)
import functools

import jax
import jax.numpy as jnp
from jax import lax
from jax.experimental import pallas as pl
from jax.experimental.pallas import tpu as pltpu
from jax.experimental.pallas import tpu_sc as plsc


def _build(B, L, SL, N):
    NC, NS = 2, 16
    NW = NC * NS
    assert B % NW == 0
    rows_per_worker = B // NW
    RB = 16
    assert rows_per_worker % RB == 0
    chunks = rows_per_worker // RB
    WB = L // 32
    WSH = (L // 32).bit_length() - 1
    JV = SL // 16
    NBUF = 8
    IQ = N // 4

    mesh = plsc.VectorSubcoreMesh(core_axis_name="c", subcore_axis_name="s")

    @functools.partial(
        pl.kernel,
        out_type=jax.ShapeDtypeStruct((N, B, SL), jnp.float32),
        mesh=mesh,
        scratch_types=[pltpu.VMEM((RB, L // 32), jnp.int32)] * 2
          + [pltpu.VMEM((RB, SL), jnp.float32)] * 2
          + [pltpu.VMEM((N,), jnp.int32)]
          + [pltpu.VMEM((NBUF, RB, SL), jnp.float32)]
          + [pltpu.SemaphoreType.DMA] * (NBUF + 2),
        compiler_params=pltpu.CompilerParams(needs_layout_passes=False),
    )
    def k(w_hbm, s_hbm, idx_hbm, out_hbm, wrows0, wrows1, srows0, srows1,
          svmem, ob, *all_sems):
        sems = all_sems[:NBUF]
        isems = all_sems[NBUF:]
        wid = lax.axis_index("s") * NC + lax.axis_index("c")
        pltpu.sync_copy(idx_hbm, svmem)
        lane = lax.iota(jnp.int32, 16)
        zeros16 = jnp.zeros((16,), jnp.int32)
        base0 = wid * rows_per_worker

        def compute_quad(i0, slot0, wrows, srows):
            s_vecs = [plsc.load_gather(svmem, [zeros16 + (i0 + q)])
                      for q in range(4)]

            @plsc.parallel_loop(0, JV // 2)
            def _(j):
                t16 = j * 16 + lane
                off1 = j * 16 + (JV // 2) * 16
                pws = []
                for q in range(4):
                    pos = s_vecs[q] + t16
                    bm0 = jnp.int32(1) << (pos >> WSH)
                    pws.append((pos & (WB - 1), bm0, bm0 << 1))
                for bl in range(RB):
                    v0 = srows[bl, pl.ds(j * 16, 16)]
                    v1 = srows[bl, pl.ds(off1, 16)]
                    for q in range(4):
                        w = plsc.load_gather(
                            wrows, [zeros16 + bl, pws[q][0]])
                        ob[slot0 + q, bl, pl.ds(j * 16, 16)] = jnp.where(
                            (w & pws[q][1]) != 0, v0, 0.0)
                        ob[slot0 + q, bl, pl.ds(off1, 16)] = jnp.where(
                            (w & pws[q][2]) != 0, v1, 0.0)

        pltpu.async_copy(w_hbm.at[pl.ds(base0, RB)], wrows0, isems[0])
        pltpu.async_copy(s_hbm.at[pl.ds(base0, RB)], srows0, isems[0])
        pltpu.async_copy(w_hbm.at[pl.ds(base0 + RB, RB)], wrows1, isems[1])
        pltpu.async_copy(s_hbm.at[pl.ds(base0 + RB, RB)], srows1, isems[1])

        def cp_body(cp, _):
            for half, (wr, sr, isem) in enumerate(
                    ((wrows0, srows0, isems[0]), (wrows1, srows1, isems[1]))):
                c = 2 * cp + half
                base = base0 + c * RB
                pltpu.make_async_copy(
                    w_hbm.at[pl.ds(0, RB)], wr, isem).wait()
                pltpu.make_async_copy(
                    s_hbm.at[pl.ds(0, RB)], sr, isem).wait()

                def iq_body(iq, _, c=c, base=base, wr=wr, sr=sr,
                            half=half):
                    g = iq & 1
                    for gs in range(2):
                        @pl.when(g == gs)
                        def _(gs=gs):
                            for q in range(4):
                                s0 = gs * 4
                                if half == 0:
                                    @pl.when((cp > 0) | (iq > 1))
                                    def _(q=q, s0=s0):
                                        pltpu.make_async_copy(
                                            ob.at[s0 + q],
                                            out_hbm.at[0, pl.ds(0, RB)],
                                            sems[s0 + q]).wait()
                                else:
                                    pltpu.make_async_copy(
                                        ob.at[s0 + q],
                                        out_hbm.at[0, pl.ds(0, RB)],
                                        sems[s0 + q]).wait()
                    compute_quad(4 * iq, g * 4, wr, sr)
                    for gs in range(2):
                        @pl.when(g == gs)
                        def _(gs=gs):
                            for q in range(4):
                                pltpu.async_copy(
                                    ob.at[gs * 4 + q],
                                    out_hbm.at[4 * iq + q, pl.ds(base, RB)],
                                    sems[gs * 4 + q])
                    return 0

                lax.fori_loop(0, IQ, iq_body, 0)

                @pl.when(c + 2 < chunks)
                def _(base=base, wr=wr, sr=sr, isem=isem):
                    pltpu.async_copy(
                        w_hbm.at[pl.ds(base + 2 * RB, RB)], wr, isem)
                    pltpu.async_copy(
                        s_hbm.at[pl.ds(base + 2 * RB, RB)], sr, isem)
            return 0

        lax.fori_loop(0, chunks // 2, cp_body, 0)
        for q in range(NBUF):
            pltpu.make_async_copy(
                ob.at[q], out_hbm.at[0, pl.ds(0, RB)], sems[q]).wait()

    return k


def kernel(where_input, slice_input, slice_len, start_indices):
    B, L = where_input.shape
    SL = slice_input.shape[1]
    N = start_indices.shape[0]
    zero_off = (jnp.asarray(slice_len) - SL).astype(jnp.int32)
    starts = jnp.clip(
        start_indices.astype(jnp.int32) + zero_off, 0, L - SL)
    WB = L // 32
    wbits = jnp.where(where_input[:, :WB], jnp.int32(1), jnp.int32(0))
    for kk in range(1, 32):
        wbits = wbits | jnp.where(
            where_input[:, kk * WB:(kk + 1) * WB], jnp.int32(1) << kk,
            jnp.int32(0))
    return _build(B, L, SL, N)(wbits, slice_input, starts)


# speedup vs baseline: 1.1397x; 1.1397x over previous
import functools

import jax
import jax.numpy as jnp
from jax import lax
from jax.experimental import pallas as pl
from jax.experimental.pallas import tpu as pltpu
from jax.experimental.pallas import tpu_sc as plsc


def _build(B, L, SL, N):
    NC, NS = 2, 16
    NW = NC * NS
    assert B % NW == 0
    rows_per_worker = B // NW
    RB = 16
    assert rows_per_worker % RB == 0
    chunks = rows_per_worker // RB
    assert chunks >= 2 and chunks % 2 == 0
    WB = L // 32
    WSH = WB.bit_length() - 1
    assert WB == 1 << WSH
    JV = SL // 16
    assert JV % 2 == 0 and N % 4 == 0
    IQ = N // 4

    mesh = plsc.VectorSubcoreMesh(core_axis_name="c", subcore_axis_name="s")

    @functools.partial(
        pl.kernel,
        out_type=jax.ShapeDtypeStruct((N, B, SL), jnp.float32),
        mesh=mesh,
        scratch_types=[
            pltpu.VMEM((2, RB, WB), jnp.int32),
            pltpu.VMEM((2, RB, SL), jnp.float32),
            pltpu.VMEM((N,), jnp.int32),
            pltpu.VMEM((8, RB, SL), jnp.float32),
        ] + [pltpu.SemaphoreType.DMA] * 10,
        compiler_params=pltpu.CompilerParams(needs_layout_passes=False),
    )
    def k(w_hbm, s_hbm, idx_hbm, out_hbm, wrows, srows, svmem, ob, *sems):
        osems = sems[:8]
        isems = sems[8:]
        wid = lax.axis_index("s") * NC + lax.axis_index("c")
        pltpu.sync_copy(idx_hbm, svmem)
        lane = lax.iota(jnp.int32, 16)
        zeros16 = jnp.zeros((16,), jnp.int32)
        base0 = wid * rows_per_worker

        def compute_quad(i0, slot0, slot):
            s_vecs = [plsc.load_gather(svmem, [zeros16 + (i0 + q)])
                      for q in range(4)]
            slot_v = zeros16 + slot

            @plsc.parallel_loop(0, JV // 2)
            def _(j):
                t16 = j * 16 + lane
                off1 = j * 16 + (JV // 2) * 16
                pws = []
                for q in range(4):
                    pos = s_vecs[q] + t16
                    bm0 = jnp.int32(1) << (pos >> WSH)
                    pws.append((pos & (WB - 1), bm0, bm0 << 1))
                for bl in range(RB):
                    v0 = srows[slot, bl, pl.ds(j * 16, 16)]
                    v1 = srows[slot, bl, pl.ds(off1, 16)]
                    for q in range(4):
                        w = plsc.load_gather(
                            wrows, [slot_v, zeros16 + bl, pws[q][0]])
                        ob[slot0 + q, bl, pl.ds(j * 16, 16)] = jnp.where(
                            (w & pws[q][1]) != 0, v0, 0.0)
                        ob[slot0 + q, bl, pl.ds(off1, 16)] = jnp.where(
                            (w & pws[q][2]) != 0, v1, 0.0)

        for ss in range(2):
            pltpu.async_copy(
                w_hbm.at[pl.ds(base0 + ss * RB, RB)], wrows.at[ss],
                isems[ss])
            pltpu.async_copy(
                s_hbm.at[pl.ds(base0 + ss * RB, RB)], srows.at[ss],
                isems[ss])

        def c_body(c, _):
            slot = c & 1
            base = base0 + c * RB

            for ss in range(2):
                @pl.when(slot == ss)
                def _(ss=ss):
                    pltpu.make_async_copy(
                        w_hbm.at[pl.ds(0, RB)], wrows.at[ss],
                        isems[ss]).wait()
                    pltpu.make_async_copy(
                        s_hbm.at[pl.ds(0, RB)], srows.at[ss],
                        isems[ss]).wait()

            def iq_body(iq, _):
                g = iq & 1
                for gs in range(2):
                    @pl.when((g == gs) & ((c > 0) | (iq > 1)))
                    def _(gs=gs):
                        for q in range(4):
                            pltpu.make_async_copy(
                                ob.at[gs * 4 + q],
                                out_hbm.at[0, pl.ds(0, RB)],
                                osems[gs * 4 + q]).wait()
                compute_quad(4 * iq, g * 4, slot)
                for gs in range(2):
                    @pl.when(g == gs)
                    def _(gs=gs):
                        for q in range(4):
                            pltpu.async_copy(
                                ob.at[gs * 4 + q],
                                out_hbm.at[4 * iq + q, pl.ds(base, RB)],
                                osems[gs * 4 + q])
                return 0

            lax.fori_loop(0, IQ, iq_body, 0)

            @pl.when(c + 2 < chunks)
            def _():
                for ss in range(2):
                    @pl.when(slot == ss)
                    def _(ss=ss):
                        pltpu.async_copy(
                            w_hbm.at[pl.ds(base + 2 * RB, RB)],
                            wrows.at[ss], isems[ss])
                        pltpu.async_copy(
                            s_hbm.at[pl.ds(base + 2 * RB, RB)],
                            srows.at[ss], isems[ss])
            return 0

        lax.fori_loop(0, chunks, c_body, 0)
        for q in range(8):
            pltpu.make_async_copy(
                ob.at[q], out_hbm.at[0, pl.ds(0, RB)], osems[q]).wait()

    return k


def kernel(where_input, slice_input, slice_len, start_indices):
    B, L = where_input.shape
    SL = slice_input.shape[1]
    N = start_indices.shape[0]
    zero_off = (jnp.asarray(slice_len) - SL).astype(jnp.int32)
    starts = jnp.clip(
        start_indices.astype(jnp.int32) + zero_off, 0, L - SL)
    WB = L // 32
    wbits = jnp.where(where_input[:, :WB], jnp.int32(1), jnp.int32(0))
    for kk in range(1, 32):
        wbits = wbits | jnp.where(
            where_input[:, kk * WB:(kk + 1) * WB], jnp.int32(1) << kk,
            jnp.int32(0))
    return _build(B, L, SL, N)(wbits, slice_input, starts)


# speedup vs baseline: 2.4489x; 2.1487x over previous
import functools

import jax
import jax.numpy as jnp
from jax import lax
from jax.experimental import pallas as pl
from jax.experimental.pallas import tpu as pltpu
from jax.experimental.pallas import tpu_sc as plsc


def _build(B, L, SL, N):
    NC, NS = 2, 16
    NW = NC * NS
    assert B % NW == 0
    rows_per_worker = B // NW
    RB = 16
    assert rows_per_worker % RB == 0
    chunks = rows_per_worker // RB
    WB = L // 32
    WSH = (L // 32).bit_length() - 1
    JV = SL // 16
    NBUF = 4
    IQ = N // 2

    mesh = plsc.VectorSubcoreMesh(core_axis_name="c", subcore_axis_name="s")

    @functools.partial(
        pl.kernel,
        out_type=jax.ShapeDtypeStruct((N, B, SL), jnp.float32),
        mesh=mesh,
        scratch_types=[pltpu.VMEM((RB, L // 32), jnp.int32)] * 2
          + [pltpu.VMEM((RB, SL), jnp.float32)] * 2
          + [pltpu.VMEM((N,), jnp.int32)]
          + [pltpu.VMEM((RB, SL), jnp.float32)] * NBUF
          + [pltpu.SemaphoreType.DMA] * (NBUF + 2),
        compiler_params=pltpu.CompilerParams(needs_layout_passes=False),
    )
    def k(w_hbm, s_hbm, idx_hbm, out_hbm, wrows0, wrows1, srows0, srows1,
          svmem, *obs_sems):
        obs = obs_sems[:NBUF]
        sems = obs_sems[NBUF:NBUF * 2]
        isems = obs_sems[NBUF * 2:]
        wid = lax.axis_index("s") * NC + lax.axis_index("c")
        pltpu.sync_copy(idx_hbm, svmem)
        lane = lax.iota(jnp.int32, 16)
        zeros16 = jnp.zeros((16,), jnp.int32)
        base0 = wid * rows_per_worker

        def compute_pair(i0, obA, obB, wrows, srows):
            sA = plsc.load_gather(svmem, [zeros16 + i0])
            sB = plsc.load_gather(svmem, [zeros16 + (i0 + 1)])

            @plsc.parallel_loop(0, JV // 2)
            def _(j):
                t16 = j * 16 + lane
                off1 = j * 16 + (JV // 2) * 16
                posA = sA + t16
                wxA = posA & (WB - 1)
                bA0 = jnp.int32(1) << (posA >> WSH)
                bA1 = bA0 << 1
                posB = sB + t16
                wxB = posB & (WB - 1)
                bB0 = jnp.int32(1) << (posB >> WSH)
                bB1 = bB0 << 1
                for bl in range(RB):
                    v0 = srows[bl, pl.ds(j * 16, 16)]
                    v1 = srows[bl, pl.ds(off1, 16)]
                    wA = plsc.load_gather(wrows, [zeros16 + bl, wxA])
                    obA[bl, pl.ds(j * 16, 16)] = jnp.where(
                        (wA & bA0) != 0, v0, 0.0)
                    obA[bl, pl.ds(off1, 16)] = jnp.where(
                        (wA & bA1) != 0, v1, 0.0)
                    wB = plsc.load_gather(wrows, [zeros16 + bl, wxB])
                    obB[bl, pl.ds(j * 16, 16)] = jnp.where(
                        (wB & bB0) != 0, v0, 0.0)
                    obB[bl, pl.ds(off1, 16)] = jnp.where(
                        (wB & bB1) != 0, v1, 0.0)

        pltpu.async_copy(w_hbm.at[pl.ds(base0, RB)], wrows0, isems[0])
        pltpu.async_copy(s_hbm.at[pl.ds(base0, RB)], srows0, isems[0])
        pltpu.async_copy(w_hbm.at[pl.ds(base0 + RB, RB)], wrows1, isems[1])
        pltpu.async_copy(s_hbm.at[pl.ds(base0 + RB, RB)], srows1, isems[1])

        def cp_body(cp, _):
            for half, (wr, sr, isem) in enumerate(
                    ((wrows0, srows0, isems[0]), (wrows1, srows1, isems[1]))):
                c = 2 * cp + half
                base = base0 + c * RB
                pltpu.make_async_copy(
                    w_hbm.at[pl.ds(0, RB)], wr, isem).wait()
                pltpu.make_async_copy(
                    s_hbm.at[pl.ds(0, RB)], sr, isem).wait()

                def iq_body(iq2, _, c=c, base=base, wr=wr, sr=sr,
                            half=half):
                    for g in range(2):
                        iq = 2 * iq2 + g
                        bA, bB = obs[2 * g], obs[2 * g + 1]
                        sA_, sB_ = sems[2 * g], sems[2 * g + 1]
                        if half == 0:
                            @pl.when((cp > 0) | (iq2 > 0))
                            def _(bA=bA, bB=bB, sA_=sA_, sB_=sB_):
                                pltpu.make_async_copy(
                                    bA, out_hbm.at[0, pl.ds(0, RB)],
                                    sA_).wait()
                                pltpu.make_async_copy(
                                    bB, out_hbm.at[0, pl.ds(0, RB)],
                                    sB_).wait()
                        else:
                            pltpu.make_async_copy(
                                bA, out_hbm.at[0, pl.ds(0, RB)], sA_).wait()
                            pltpu.make_async_copy(
                                bB, out_hbm.at[0, pl.ds(0, RB)], sB_).wait()
                        compute_pair(2 * iq, bA, bB, wr, sr)
                        pltpu.async_copy(
                            bA, out_hbm.at[2 * iq, pl.ds(base, RB)], sA_)
                        pltpu.async_copy(
                            bB, out_hbm.at[2 * iq + 1, pl.ds(base, RB)],
                            sB_)
                    return 0

                lax.fori_loop(0, IQ // 2, iq_body, 0)

                @pl.when(c + 2 < chunks)
                def _(base=base, wr=wr, sr=sr, isem=isem):
                    pltpu.async_copy(
                        w_hbm.at[pl.ds(base + 2 * RB, RB)], wr, isem)
                    pltpu.async_copy(
                        s_hbm.at[pl.ds(base + 2 * RB, RB)], sr, isem)
            return 0

        lax.fori_loop(0, chunks // 2, cp_body, 0)
        for q in range(NBUF):
            pltpu.make_async_copy(
                obs[q], out_hbm.at[0, pl.ds(0, RB)], sems[q]).wait()

    return k


def kernel(where_input, slice_input, slice_len, start_indices):
    B, L = where_input.shape
    SL = slice_input.shape[1]
    N = start_indices.shape[0]
    zero_off = (jnp.asarray(slice_len) - SL).astype(jnp.int32)
    starts = jnp.clip(
        start_indices.astype(jnp.int32) + zero_off, 0, L - SL)
    WB = L // 32
    wbits = jnp.where(where_input[:, :WB], jnp.int32(1), jnp.int32(0))
    for kk in range(1, 32):
        wbits = wbits | jnp.where(
            where_input[:, kk * WB:(kk + 1) * WB], jnp.int32(1) << kk,
            jnp.int32(0))
    return _build(B, L, SL, N)(wbits, slice_input, starts)
